# baseline (speedup 1.0000x reference)
import functools

import jax
import jax.numpy as jnp
from jax import lax
from jax.experimental import pallas as pl
from jax.experimental.pallas import tpu as pltpu
from jax.experimental.pallas import tpu_sc as plsc

F = 26
V = 100000
L = 16
B = 16384
H = 64

NC = 2
NS = 16
NW = NC * NS
BPW = B // NW
CH = 64
IDX_PER_CH = CH * F
IDX_ROWS = IDX_PER_CH // 128
CHUNKS = BPW // CH


def _gather_sum_body(emb_hbm, idx_hbm, out_hbm, idx_v, rows_v, h_v, sem):
    wid = lax.axis_index("s") * NC + lax.axis_index("c")

    pltpu.sync_copy(idx_hbm.at[pl.ds(wid * (IDX_ROWS * CHUNKS), IDX_ROWS * CHUNKS)],
                    idx_v)

    def chunk_body(g, carry):
        copies = [
            pltpu.async_copy(
                emb_hbm.at[idx_v.at[g * IDX_ROWS + j]],
                rows_v.at[pl.ds(j * 128, 128)],
                sem,
            )
            for j in range(IDX_ROWS)
        ]
        for c in copies:
            c.wait()

        def sum_body(b, carry2):
            base = b * F
            acc = rows_v[base]
            for f in range(1, F):
                acc = acc + rows_v[base + f]
            h_v[b] = acc
            return carry2

        lax.fori_loop(0, CH, sum_body, 0, unroll=2)

        pltpu.sync_copy(h_v, out_hbm.at[pl.ds(wid * BPW + g * CH, CH)])
        return carry

    lax.fori_loop(0, CHUNKS, chunk_body, 0)


def _gather_sum(emb_flat, idx2d):
    mesh = plsc.VectorSubcoreMesh(
        core_axis_name="c", subcore_axis_name="s", num_cores=NC, num_subcores=NS)
    return pl.kernel(
        _gather_sum_body,
        out_type=jax.ShapeDtypeStruct((B, L), jnp.float32),
        mesh=mesh,
        scratch_types=[
            pltpu.VMEM((IDX_ROWS * CHUNKS, 128), jnp.int32),
            pltpu.VMEM((IDX_PER_CH, L), jnp.float32),
            pltpu.VMEM((CH, L), jnp.float32),
            pltpu.SemaphoreType.DMA,
        ],
        compiler_params=pltpu.CompilerParams(use_tc_tiling_on_sc=False),
    )(emb_flat, idx2d)


MLP_BLK = 2048


def _mlp_body(h_ref, w1_ref, b1_ref, w2_ref, b2_ref, w3_ref, b3_ref, out_ref):
    x = h_ref[...]
    x = jnp.maximum(
        jnp.dot(x, w1_ref[...], preferred_element_type=jnp.float32) + b1_ref[...], 0.0)
    x = jnp.maximum(
        jnp.dot(x, w2_ref[...], preferred_element_type=jnp.float32) + b2_ref[...], 0.0)
    out_ref[...] = jnp.maximum(
        jnp.dot(x, w3_ref[...], preferred_element_type=jnp.float32) + b3_ref[...], 0.0)


def _mlp(h, W1, b1, W2, b2, W3, b3):
    full = lambda s: pl.BlockSpec(s, lambda i: (0, 0))
    return pl.pallas_call(
        _mlp_body,
        grid=(B // MLP_BLK,),
        in_specs=[
            pl.BlockSpec((MLP_BLK, L), lambda i: (i, 0)),
            full(W1.shape), full((1, L)),
            full(W2.shape), full((1, 2 * L)),
            full(W3.shape), full((1, H)),
        ],
        out_specs=pl.BlockSpec((MLP_BLK, H), lambda i: (i, 0)),
        out_shape=jax.ShapeDtypeStruct((B, H), jnp.float32),
    )(h, W1, b1.reshape(1, L), W2, b2.reshape(1, 2 * L), W3, b3.reshape(1, H))


def kernel(mof, emb, W1, b1, W2, b2, W3, b3):
    offs = (jnp.arange(F, dtype=jnp.int32) * V)[None, :]
    flat_idx = (mof.astype(jnp.int32) + offs).reshape(-1, 128)
    emb_flat = emb.reshape(F * V, L)
    h = _gather_sum(emb_flat, flat_idx)
    return _mlp(h, W1, b1, W2, b2, W3, b3)
